# baseline (speedup 1.0000x reference)
import jax
import jax.numpy as jnp
from jax import lax
from jax.experimental import pallas as pl
from jax.experimental.pallas import tpu as pltpu
from jax.experimental.pallas import tpu_sc as plsc

_N = 10000
_E = 320000
_D = 128
_NC = 2
_NS = 16
_TILES = _NC * _NS
_B = 128
_NB = 80
_EP = _TILES * _NB * _B
_SLOTS = 1
_NP = 10240
_RPT = _NP // _NS
_ZB = _B
_VPR = _D // 16


def _spmm_body(h_hbm, edata_hbm, vals_hbm, out_hbm, *refs):
    ed_s = refs[0:_SLOTS]
    vals_s = refs[_SLOTS:2 * _SLOTS]
    msg_s = refs[2 * _SLOTS:3 * _SLOTS]
    acc_sh = refs[3 * _SLOTS]
    sems = refs[3 * _SLOTS + 1:3 * _SLOTS + 1 + _SLOTS]
    ssems = refs[3 * _SLOTS + 1 + _SLOTS:3 * _SLOTS + 1 + 2 * _SLOTS]

    c = lax.axis_index("c")
    s = lax.axis_index("s")
    tid = c * _NS + s

    zbuf = msg_s[0]
    def _zrow(i, carry):
        for j in range(_VPR):
            zbuf[i, pl.ds(j * 16, 16)] = jnp.zeros((16,), jnp.float32)
        return carry
    lax.fori_loop(0, _ZB, _zrow, 0)
    for k in range(_RPT // _ZB):
        pltpu.sync_copy(zbuf, acc_sh.at[pl.ds(s * _RPT + k * _ZB, _ZB)])
    plsc.subcore_barrier()

    ebase = tid * _NB

    def _prime(slot, b, first):
        ed = ed_s[slot]
        pltpu.sync_copy(edata_hbm.at[ebase + b], ed)
        pltpu.sync_copy(vals_hbm.at[ebase + b], vals_s[slot])
        if not first:
            pltpu.make_async_copy(
                msg_s[slot], acc_sh.at[ed.at[1]], ssems[slot]).wait()
        pltpu.async_copy(h_hbm.at[ed.at[0]], msg_s[slot], sems[slot])

    def _proc(slot):
        ed = ed_s[slot]
        buf = msg_s[slot]
        pltpu.make_async_copy(h_hbm.at[ed.at[0]], buf, sems[slot]).wait()

        def _scale(g, carry):
            vv = vals_s[slot][0, pl.ds(g * 16, 16)]
            for k in range(16):
                v = vv[k]
                r = g * 16 + k
                for j in range(_VPR):
                    sl = pl.ds(j * 16, 16)
                    buf[r, sl] = buf[r, sl] * v
            return carry
        lax.fori_loop(0, _B // 16, _scale, 0)
        pltpu.async_copy(buf, acc_sh.at[ed.at[1]], ssems[slot], add=True)

    _prime(0, 0, True)

    def _round(i, carry):
        _proc(0)
        _prime(0, i + 1, False)
        return carry
    lax.fori_loop(0, _NB - 1, _round, 0)
    _proc(0)
    pltpu.make_async_copy(msg_s[0], acc_sh.at[ed_s[0].at[1]], ssems[0]).wait()

    plsc.subcore_barrier()
    for k in range(_RPT // _ZB):
        r0 = s * _RPT + k * _ZB
        pltpu.sync_copy(acc_sh.at[pl.ds(r0, _ZB)], zbuf)
        pltpu.sync_copy(zbuf, out_hbm.at[c, pl.ds(r0, _ZB)])


def _spmm(h, edata, valsb):
    mesh = plsc.VectorSubcoreMesh(
        core_axis_name="c", subcore_axis_name="s",
        num_cores=_NC, num_subcores=_NS)
    scratch = (
        [pltpu.VMEM((2, _B), jnp.int32) for _ in range(_SLOTS)]
        + [pltpu.VMEM((1, _B), jnp.float32) for _ in range(_SLOTS)]
        + [pltpu.VMEM((_B, _D), jnp.float32) for _ in range(_SLOTS)]
        + [pltpu.VMEM_SHARED((_NP, _D), jnp.float32)]
        + [pltpu.SemaphoreType.DMA for _ in range(2 * _SLOTS)]
    )
    return pl.kernel(
        _spmm_body,
        out_type=jax.ShapeDtypeStruct((_NC, _NP, _D), jnp.float32),
        mesh=mesh,
        scratch_types=scratch,
    )(h, edata, valsb)


_BLK = 1000


def _layer_body(xa, xb, w, b, o):
    x = xa[0] + xb[0]
    y = jnp.dot(x, w[...], preferred_element_type=jnp.float32) + b[...]
    o[...] = jnp.where(y >= 0, y, 0.2 * y)


def _layer(parts, w, b):
    return pl.pallas_call(
        _layer_body,
        grid=(_N // _BLK,),
        in_specs=[
            pl.BlockSpec((1, _BLK, _D), lambda i: (0, i, 0)),
            pl.BlockSpec((1, _BLK, _D), lambda i: (1, i, 0)),
            pl.BlockSpec((_D, _D), lambda i: (0, 0)),
            pl.BlockSpec((1, _D), lambda i: (0, 0)),
        ],
        out_specs=pl.BlockSpec((_BLK, _D), lambda i: (i, 0)),
        out_shape=jax.ShapeDtypeStruct((_N, _D), jnp.float32),
    )(parts, parts, w, b.reshape(1, _D))


def _final_body(xa, xb, w1, b1, wmu, bmu, wlv, blv, ini,
                tuned_o, mu_o, lv_o):
    x = xa[0] + xb[0]
    h = jnp.dot(x, w1[...], preferred_element_type=jnp.float32) + b1[...]
    h = jnp.where(h >= 0, h, 0.2 * h)
    mu = jnp.dot(h, wmu[...], preferred_element_type=jnp.float32) + bmu[...]
    lv = jnp.dot(h, wlv[...], preferred_element_type=jnp.float32) + blv[...]
    mu_o[...] = mu
    lv_o[...] = jnp.clip(lv, -20.0, 20.0)
    tuned_o[...] = ini[...] + jnp.where(mu >= 0, mu, 0.25 * mu)


def _final(parts, w1, b1, wmu, bmu, wlv, blv, ini):
    full = pl.BlockSpec((_D, _D), lambda i: (0, 0))
    vec = pl.BlockSpec((1, _D), lambda i: (0, 0))
    blk = pl.BlockSpec((_BLK, _D), lambda i: (i, 0))
    return pl.pallas_call(
        _final_body,
        grid=(_N // _BLK,),
        in_specs=[
            pl.BlockSpec((1, _BLK, _D), lambda i: (0, i, 0)),
            pl.BlockSpec((1, _BLK, _D), lambda i: (1, i, 0)),
            full, vec, full, vec, full, vec, blk,
        ],
        out_specs=(blk, blk, blk),
        out_shape=(
            jax.ShapeDtypeStruct((_N, _D), jnp.float32),
            jax.ShapeDtypeStruct((_N, _D), jnp.float32),
            jax.ShapeDtypeStruct((_N, _D), jnp.float32),
        ),
    )(parts, parts, w1, b1.reshape(1, _D), wmu, bmu.reshape(1, _D),
      wlv, blv.reshape(1, _D), ini)


@jax.jit
def kernel(edge_index, edge_vals, node_feats, ini_embeds,
           W0, b0, W1, b1, Wmu, bmu, Wlv, blv):
    pad = _EP - _E
    rows_p = jnp.concatenate([edge_index[0], jnp.full((pad,), _N, jnp.int32)])
    cols_p = jnp.concatenate([edge_index[1], jnp.zeros((pad,), jnp.int32)])
    vals_p = jnp.concatenate([edge_vals, jnp.zeros((pad,), jnp.float32)])
    edata = jnp.stack(
        [cols_p.reshape(-1, _B), rows_p.reshape(-1, _B)], axis=1)
    valsb = vals_p.reshape(-1, 1, _B)

    s1 = _spmm(node_feats, edata, valsb)
    h1 = _layer(s1, W0, b0)
    s2 = _spmm(h1, edata, valsb)
    return _final(s2, W1, b1, Wmu, bmu, Wlv, blv, ini_embeds)


# speedup vs baseline: 1.6827x; 1.6827x over previous
import jax
import jax.numpy as jnp
from jax import lax
from jax.experimental import pallas as pl
from jax.experimental.pallas import tpu as pltpu
from jax.experimental.pallas import tpu_sc as plsc

_N = 10000
_E = 320000
_D = 128
_NC = 2
_NS = 16
_TILES = _NC * _NS
_EPT = _E // _TILES
_B = 128
_NFULL = _EPT // _B
_RTAIL = _EPT - _NFULL * _B
_NP = 10240
_RPT = _NP // _NS
_ZR = 128
_VPR = _D // 16


def _spmm_body(h_hbm, rows_hbm, cols_hbm, vals_hbm, out_hbm,
               idx_v, ridx_v, vals_v, msg_v,
               idx_t, ridx_t, vals_t, msg_t,
               zbuf_v, acc_sh, sem):
    c = lax.axis_index("c")
    s = lax.axis_index("s")
    tid = c * _NS + s

    def _zrow(i, carry):
        for j in range(_VPR):
            zbuf_v[i, pl.ds(j * 16, 16)] = jnp.zeros((16,), jnp.float32)
        return carry
    lax.fori_loop(0, _ZR, _zrow, 0)
    for k in range(_RPT // _ZR):
        pltpu.sync_copy(zbuf_v, acc_sh.at[pl.ds(s * _RPT + k * _ZR, _ZR)])
    plsc.subcore_barrier()

    ebase = tid * _EPT

    def _do_batch(base, nb, idx, ridx, vals, msg):
        pltpu.sync_copy(cols_hbm.at[pl.ds(base, nb)], idx)
        pltpu.sync_copy(rows_hbm.at[pl.ds(base, nb)], ridx)
        pltpu.sync_copy(vals_hbm.at[pl.ds(base, nb)], vals)
        pltpu.async_copy(h_hbm.at[idx], msg, sem).wait()

        def _scale(g, carry):
            vv = vals[pl.ds(g * 16, 16)]
            for k in range(16):
                v = vv[k]
                r = g * 16 + k
                for j in range(_VPR):
                    sl = pl.ds(j * 16, 16)
                    msg[r, sl] = msg[r, sl] * v
            return carry
        lax.fori_loop(0, nb // 16, _scale, 0)
        pltpu.sync_copy(msg, acc_sh.at[ridx], add=True)

    def _batch(b, carry):
        _do_batch(ebase + b * _B, _B, idx_v, ridx_v, vals_v, msg_v)
        return carry
    lax.fori_loop(0, _NFULL, _batch, 0)
    _do_batch(ebase + _NFULL * _B, _RTAIL, idx_t, ridx_t, vals_t, msg_t)

    plsc.subcore_barrier()
    for k in range(_RPT // _ZR):
        r0 = s * _RPT + k * _ZR
        pltpu.sync_copy(acc_sh.at[pl.ds(r0, _ZR)], zbuf_v)
        pltpu.sync_copy(zbuf_v, out_hbm.at[c, pl.ds(r0, _ZR)])


def _spmm(h, rows, cols, vals):
    mesh = plsc.VectorSubcoreMesh(
        core_axis_name="c", subcore_axis_name="s",
        num_cores=_NC, num_subcores=_NS)
    return pl.kernel(
        _spmm_body,
        out_type=jax.ShapeDtypeStruct((_NC, _NP, _D), jnp.float32),
        mesh=mesh,
        scratch_types=[
            pltpu.VMEM((_B,), jnp.int32),
            pltpu.VMEM((_B,), jnp.int32),
            pltpu.VMEM((_B,), jnp.float32),
            pltpu.VMEM((_B, _D), jnp.float32),
            pltpu.VMEM((_RTAIL,), jnp.int32),
            pltpu.VMEM((_RTAIL,), jnp.int32),
            pltpu.VMEM((_RTAIL,), jnp.float32),
            pltpu.VMEM((_RTAIL, _D), jnp.float32),
            pltpu.VMEM((_ZR, _D), jnp.float32),
            pltpu.VMEM_SHARED((_NP, _D), jnp.float32),
            pltpu.SemaphoreType.DMA,
        ],
    )(h, rows, cols, vals)


_BLK = 1000


def _layer_body(xa, xb, w, b, o):
    x = xa[0] + xb[0]
    y = jnp.dot(x, w[...], preferred_element_type=jnp.float32) + b[...]
    o[...] = jnp.where(y >= 0, y, 0.2 * y)


def _layer(parts, w, b):
    return pl.pallas_call(
        _layer_body,
        grid=(_N // _BLK,),
        in_specs=[
            pl.BlockSpec((1, _BLK, _D), lambda i: (0, i, 0)),
            pl.BlockSpec((1, _BLK, _D), lambda i: (1, i, 0)),
            pl.BlockSpec((_D, _D), lambda i: (0, 0)),
            pl.BlockSpec((1, _D), lambda i: (0, 0)),
        ],
        out_specs=pl.BlockSpec((_BLK, _D), lambda i: (i, 0)),
        out_shape=jax.ShapeDtypeStruct((_N, _D), jnp.float32),
    )(parts, parts, w, b.reshape(1, _D))


def _final_body(xa, xb, w1, b1, wmu, bmu, wlv, blv, ini,
                tuned_o, mu_o, lv_o):
    x = xa[0] + xb[0]
    h = jnp.dot(x, w1[...], preferred_element_type=jnp.float32) + b1[...]
    h = jnp.where(h >= 0, h, 0.2 * h)
    mu = jnp.dot(h, wmu[...], preferred_element_type=jnp.float32) + bmu[...]
    lv = jnp.dot(h, wlv[...], preferred_element_type=jnp.float32) + blv[...]
    mu_o[...] = mu
    lv_o[...] = jnp.clip(lv, -20.0, 20.0)
    tuned_o[...] = ini[...] + jnp.where(mu >= 0, mu, 0.25 * mu)


def _final(parts, w1, b1, wmu, bmu, wlv, blv, ini):
    full = pl.BlockSpec((_D, _D), lambda i: (0, 0))
    vec = pl.BlockSpec((1, _D), lambda i: (0, 0))
    blk = pl.BlockSpec((_BLK, _D), lambda i: (i, 0))
    return pl.pallas_call(
        _final_body,
        grid=(_N // _BLK,),
        in_specs=[
            pl.BlockSpec((1, _BLK, _D), lambda i: (0, i, 0)),
            pl.BlockSpec((1, _BLK, _D), lambda i: (1, i, 0)),
            full, vec, full, vec, full, vec, blk,
        ],
        out_specs=(blk, blk, blk),
        out_shape=(
            jax.ShapeDtypeStruct((_N, _D), jnp.float32),
            jax.ShapeDtypeStruct((_N, _D), jnp.float32),
            jax.ShapeDtypeStruct((_N, _D), jnp.float32),
        ),
    )(parts, parts, w1, b1.reshape(1, _D), wmu, bmu.reshape(1, _D),
      wlv, blv.reshape(1, _D), ini)


@jax.jit
def kernel(edge_index, edge_vals, node_feats, ini_embeds,
           W0, b0, W1, b1, Wmu, bmu, Wlv, blv):
    rows = edge_index[0]
    cols = edge_index[1]
    s1 = _spmm(node_feats, rows, cols, edge_vals)
    h1 = _layer(s1, W0, b0)
    s2 = _spmm(h1, rows, cols, edge_vals)
    return _final(s2, W1, b1, Wmu, bmu, Wlv, blv, ini_embeds)


# speedup vs baseline: 2.2801x; 1.3550x over previous
import jax
import jax.numpy as jnp
from jax import lax
from jax.experimental import pallas as pl
from jax.experimental.pallas import tpu as pltpu
from jax.experimental.pallas import tpu_sc as plsc

_N = 10000
_E = 320000
_D = 128
_NC = 2
_NS = 16
_TILES = _NC * _NS
_EPT = _E // _TILES
_B = 128
_NFULL = _EPT // _B
_RTAIL = _EPT - _NFULL * _B
_NP = 10240
_RPT = _NP // _NS
_ZR = 128
_VPR = _D // 16


def _spmm_body(h_hbm, rows_hbm, cols_hbm, vals_hbm, out_hbm,
               idx_a, ridx_a, vals_a, idx_b, ridx_b, vals_b, msg_v,
               idx_t, ridx_t, vals_t, msg_t,
               zbuf_v, acc_sh, sem):
    c = lax.axis_index("c")
    s = lax.axis_index("s")
    tid = c * _NS + s

    def _zrow(i, carry):
        for j in range(_VPR):
            zbuf_v[i, pl.ds(j * 16, 16)] = jnp.zeros((16,), jnp.float32)
        return carry
    lax.fori_loop(0, _ZR, _zrow, 0)
    for k in range(_RPT // _ZR):
        pltpu.sync_copy(zbuf_v, acc_sh.at[pl.ds(s * _RPT + k * _ZR, _ZR)])
    plsc.subcore_barrier()

    ebase = tid * _EPT

    def _copy_idx(b, idx, ridx, vals):
        base = ebase + b * _B
        pltpu.sync_copy(cols_hbm.at[pl.ds(base, _B)], idx)
        pltpu.sync_copy(rows_hbm.at[pl.ds(base, _B)], ridx)
        pltpu.sync_copy(vals_hbm.at[pl.ds(base, _B)], vals)

    def _scale_buf(vals, msg, nb):
        def _scale(g, carry):
            vv = vals[pl.ds(g * 16, 16)]
            for k in range(16):
                v = vv[k]
                r = g * 16 + k
                for j in range(_VPR):
                    sl = pl.ds(j * 16, 16)
                    msg[r, sl] = msg[r, sl] * v
            return carry
        lax.fori_loop(0, nb // 16, _scale, 0)

    def _work(idx, ridx, vals, nextb, nidx, nridx, nvals):
        pltpu.async_copy(h_hbm.at[idx], msg_v, sem)
        _copy_idx(nextb, nidx, nridx, nvals)
        pltpu.make_async_copy(h_hbm.at[idx], msg_v, sem).wait()
        _scale_buf(vals, msg_v, _B)
        pltpu.sync_copy(msg_v, acc_sh.at[ridx], add=True)

    _copy_idx(0, idx_a, ridx_a, vals_a)

    def _pair(i, carry):
        b0 = 2 * i
        _work(idx_a, ridx_a, vals_a, b0 + 1, idx_b, ridx_b, vals_b)
        b2 = jnp.minimum(b0 + 2, _NFULL - 1)
        _work(idx_b, ridx_b, vals_b, b2, idx_a, ridx_a, vals_a)
        return carry
    lax.fori_loop(0, _NFULL // 2, _pair, 0)

    tbase = ebase + _NFULL * _B
    pltpu.sync_copy(cols_hbm.at[pl.ds(tbase, _RTAIL)], idx_t)
    pltpu.sync_copy(rows_hbm.at[pl.ds(tbase, _RTAIL)], ridx_t)
    pltpu.sync_copy(vals_hbm.at[pl.ds(tbase, _RTAIL)], vals_t)
    pltpu.async_copy(h_hbm.at[idx_t], msg_t, sem).wait()
    _scale_buf(vals_t, msg_t, _RTAIL)
    pltpu.sync_copy(msg_t, acc_sh.at[ridx_t], add=True)

    plsc.subcore_barrier()
    for k in range(_RPT // _ZR):
        r0 = s * _RPT + k * _ZR
        pltpu.sync_copy(acc_sh.at[pl.ds(r0, _ZR)], zbuf_v)
        pltpu.sync_copy(zbuf_v, out_hbm.at[c, pl.ds(r0, _ZR)])


def _spmm(h, rows, cols, vals):
    mesh = plsc.VectorSubcoreMesh(
        core_axis_name="c", subcore_axis_name="s",
        num_cores=_NC, num_subcores=_NS)
    return pl.kernel(
        _spmm_body,
        out_type=jax.ShapeDtypeStruct((_NC, _NP, _D), jnp.float32),
        mesh=mesh,
        scratch_types=[
            pltpu.VMEM((_B,), jnp.int32),
            pltpu.VMEM((_B,), jnp.int32),
            pltpu.VMEM((_B,), jnp.float32),
            pltpu.VMEM((_B,), jnp.int32),
            pltpu.VMEM((_B,), jnp.int32),
            pltpu.VMEM((_B,), jnp.float32),
            pltpu.VMEM((_B, _D), jnp.float32),
            pltpu.VMEM((_RTAIL,), jnp.int32),
            pltpu.VMEM((_RTAIL,), jnp.int32),
            pltpu.VMEM((_RTAIL,), jnp.float32),
            pltpu.VMEM((_RTAIL, _D), jnp.float32),
            pltpu.VMEM((_ZR, _D), jnp.float32),
            pltpu.VMEM_SHARED((_NP, _D), jnp.float32),
            pltpu.SemaphoreType.DMA,
        ],
    )(h, rows, cols, vals)


_BLK = 1000


def _layer_body(xa, xb, w, b, o):
    x = xa[0] + xb[0]
    y = jnp.dot(x, w[...], preferred_element_type=jnp.float32) + b[...]
    o[...] = jnp.where(y >= 0, y, 0.2 * y)


def _layer(parts, w, b):
    return pl.pallas_call(
        _layer_body,
        grid=(_N // _BLK,),
        in_specs=[
            pl.BlockSpec((1, _BLK, _D), lambda i: (0, i, 0)),
            pl.BlockSpec((1, _BLK, _D), lambda i: (1, i, 0)),
            pl.BlockSpec((_D, _D), lambda i: (0, 0)),
            pl.BlockSpec((1, _D), lambda i: (0, 0)),
        ],
        out_specs=pl.BlockSpec((_BLK, _D), lambda i: (i, 0)),
        out_shape=jax.ShapeDtypeStruct((_N, _D), jnp.float32),
    )(parts, parts, w, b.reshape(1, _D))


def _final_body(xa, xb, w1, b1, wmu, bmu, wlv, blv, ini,
                tuned_o, mu_o, lv_o):
    x = xa[0] + xb[0]
    h = jnp.dot(x, w1[...], preferred_element_type=jnp.float32) + b1[...]
    h = jnp.where(h >= 0, h, 0.2 * h)
    mu = jnp.dot(h, wmu[...], preferred_element_type=jnp.float32) + bmu[...]
    lv = jnp.dot(h, wlv[...], preferred_element_type=jnp.float32) + blv[...]
    mu_o[...] = mu
    lv_o[...] = jnp.clip(lv, -20.0, 20.0)
    tuned_o[...] = ini[...] + jnp.where(mu >= 0, mu, 0.25 * mu)


def _final(parts, w1, b1, wmu, bmu, wlv, blv, ini):
    full = pl.BlockSpec((_D, _D), lambda i: (0, 0))
    vec = pl.BlockSpec((1, _D), lambda i: (0, 0))
    blk = pl.BlockSpec((_BLK, _D), lambda i: (i, 0))
    return pl.pallas_call(
        _final_body,
        grid=(_N // _BLK,),
        in_specs=[
            pl.BlockSpec((1, _BLK, _D), lambda i: (0, i, 0)),
            pl.BlockSpec((1, _BLK, _D), lambda i: (1, i, 0)),
            full, vec, full, vec, full, vec, blk,
        ],
        out_specs=(blk, blk, blk),
        out_shape=(
            jax.ShapeDtypeStruct((_N, _D), jnp.float32),
            jax.ShapeDtypeStruct((_N, _D), jnp.float32),
            jax.ShapeDtypeStruct((_N, _D), jnp.float32),
        ),
    )(parts, parts, w1, b1.reshape(1, _D), wmu, bmu.reshape(1, _D),
      wlv, blv.reshape(1, _D), ini)


@jax.jit
def kernel(edge_index, edge_vals, node_feats, ini_embeds,
           W0, b0, W1, b1, Wmu, bmu, Wlv, blv):
    rows = edge_index[0]
    cols = edge_index[1]
    s1 = _spmm(node_feats, rows, cols, edge_vals)
    h1 = _layer(s1, W0, b0)
    s2 = _spmm(h1, rows, cols, edge_vals)
    return _final(s2, W1, b1, Wmu, bmu, Wlv, blv, ini_embeds)


# speedup vs baseline: 2.3152x; 1.0154x over previous
import jax
import jax.numpy as jnp
from jax import lax
from jax.experimental import pallas as pl
from jax.experimental.pallas import tpu as pltpu
from jax.experimental.pallas import tpu_sc as plsc

_N = 10000
_E = 320000
_D = 128
_NC = 2
_NS = 16
_TILES = _NC * _NS
_EPT = _E // _TILES
_B = 128
_NFULL = _EPT // _B
_RTAIL = _EPT - _NFULL * _B
_NP = 10240
_RPT = _NP // _NS
_ZR = 128
_VPR = _D // 16


def _spmm_body(h_hbm, rows_hbm, cols_hbm, vals_hbm, out_hbm,
               idx_a, ridx_a, vals_a, idx_b, ridx_b, vals_b, msg_a, msg_b,
               idx_t, ridx_t, vals_t, msg_t,
               acc_sh, sem_a, sem_b):
    c = lax.axis_index("c")
    s = lax.axis_index("s")
    tid = c * _NS + s

    zbuf_v = msg_a
    def _zrow(i, carry):
        for j in range(_VPR):
            zbuf_v[i, pl.ds(j * 16, 16)] = jnp.zeros((16,), jnp.float32)
        return carry
    lax.fori_loop(0, _ZR, _zrow, 0)
    for k in range(_RPT // _ZR):
        pltpu.sync_copy(zbuf_v, acc_sh.at[pl.ds(s * _RPT + k * _ZR, _ZR)])
    plsc.subcore_barrier()

    ebase = tid * _EPT

    def _copy_idx(b, idx, ridx, vals):
        base = ebase + b * _B
        pltpu.sync_copy(cols_hbm.at[pl.ds(base, _B)], idx)
        pltpu.sync_copy(rows_hbm.at[pl.ds(base, _B)], ridx)
        pltpu.sync_copy(vals_hbm.at[pl.ds(base, _B)], vals)

    def _scale_buf(vals, msg, nb):
        def _scale(g, carry):
            vv = vals[pl.ds(g * 16, 16)]
            for k in range(16):
                v = vv[k]
                r = g * 16 + k
                for j in range(_VPR):
                    sl = pl.ds(j * 16, 16)
                    msg[r, sl] = msg[r, sl] * v
            return carry
        lax.fori_loop(0, nb // 16, _scale, 0)

    def _wait(idx, msg, sem):
        pltpu.make_async_copy(h_hbm.at[idx], msg, sem).wait()

    _copy_idx(0, idx_a, ridx_a, vals_a)
    pltpu.async_copy(h_hbm.at[idx_a], msg_a, sem_a)
    _copy_idx(1, idx_b, ridx_b, vals_b)

    def _pair(i, carry):
        b0 = 2 * i
        _wait(idx_a, msg_a, sem_a)
        pltpu.async_copy(h_hbm.at[idx_b], msg_b, sem_b)
        _scale_buf(vals_a, msg_a, _B)
        pltpu.sync_copy(msg_a, acc_sh.at[ridx_a], add=True)
        _copy_idx(jnp.minimum(b0 + 2, _NFULL - 1), idx_a, ridx_a, vals_a)
        _wait(idx_b, msg_b, sem_b)
        pltpu.async_copy(h_hbm.at[idx_a], msg_a, sem_a)
        _scale_buf(vals_b, msg_b, _B)
        pltpu.sync_copy(msg_b, acc_sh.at[ridx_b], add=True)
        _copy_idx(jnp.minimum(b0 + 3, _NFULL - 1), idx_b, ridx_b, vals_b)
        return carry
    lax.fori_loop(0, _NFULL // 2, _pair, 0)
    _wait(idx_a, msg_a, sem_a)

    tbase = ebase + _NFULL * _B
    pltpu.sync_copy(cols_hbm.at[pl.ds(tbase, _RTAIL)], idx_t)
    pltpu.sync_copy(rows_hbm.at[pl.ds(tbase, _RTAIL)], ridx_t)
    pltpu.sync_copy(vals_hbm.at[pl.ds(tbase, _RTAIL)], vals_t)
    pltpu.async_copy(h_hbm.at[idx_t], msg_t, sem_b).wait()
    _scale_buf(vals_t, msg_t, _RTAIL)
    pltpu.sync_copy(msg_t, acc_sh.at[ridx_t], add=True)

    plsc.subcore_barrier()
    for k in range(_RPT // _ZR):
        r0 = s * _RPT + k * _ZR
        pltpu.sync_copy(acc_sh.at[pl.ds(r0, _ZR)], zbuf_v)
        pltpu.sync_copy(zbuf_v, out_hbm.at[c, pl.ds(r0, _ZR)])


def _spmm(h, rows, cols, vals):
    mesh = plsc.VectorSubcoreMesh(
        core_axis_name="c", subcore_axis_name="s",
        num_cores=_NC, num_subcores=_NS)
    return pl.kernel(
        _spmm_body,
        out_type=jax.ShapeDtypeStruct((_NC, _NP, _D), jnp.float32),
        mesh=mesh,
        scratch_types=[
            pltpu.VMEM((_B,), jnp.int32),
            pltpu.VMEM((_B,), jnp.int32),
            pltpu.VMEM((_B,), jnp.float32),
            pltpu.VMEM((_B,), jnp.int32),
            pltpu.VMEM((_B,), jnp.int32),
            pltpu.VMEM((_B,), jnp.float32),
            pltpu.VMEM((_B, _D), jnp.float32),
            pltpu.VMEM((_B, _D), jnp.float32),
            pltpu.VMEM((_RTAIL,), jnp.int32),
            pltpu.VMEM((_RTAIL,), jnp.int32),
            pltpu.VMEM((_RTAIL,), jnp.float32),
            pltpu.VMEM((_RTAIL, _D), jnp.float32),
            pltpu.VMEM_SHARED((_NP, _D), jnp.float32),
            pltpu.SemaphoreType.DMA,
            pltpu.SemaphoreType.DMA,
        ],
    )(h, rows, cols, vals)


_BLK = 1000


def _layer_body(xa, xb, w, b, o):
    x = xa[0] + xb[0]
    y = jnp.dot(x, w[...], preferred_element_type=jnp.float32) + b[...]
    o[...] = jnp.where(y >= 0, y, 0.2 * y)


def _layer(parts, w, b):
    return pl.pallas_call(
        _layer_body,
        grid=(_N // _BLK,),
        in_specs=[
            pl.BlockSpec((1, _BLK, _D), lambda i: (0, i, 0)),
            pl.BlockSpec((1, _BLK, _D), lambda i: (1, i, 0)),
            pl.BlockSpec((_D, _D), lambda i: (0, 0)),
            pl.BlockSpec((1, _D), lambda i: (0, 0)),
        ],
        out_specs=pl.BlockSpec((_BLK, _D), lambda i: (i, 0)),
        out_shape=jax.ShapeDtypeStruct((_N, _D), jnp.float32),
    )(parts, parts, w, b.reshape(1, _D))


def _final_body(xa, xb, w1, b1, wmu, bmu, wlv, blv, ini,
                tuned_o, mu_o, lv_o):
    x = xa[0] + xb[0]
    h = jnp.dot(x, w1[...], preferred_element_type=jnp.float32) + b1[...]
    h = jnp.where(h >= 0, h, 0.2 * h)
    mu = jnp.dot(h, wmu[...], preferred_element_type=jnp.float32) + bmu[...]
    lv = jnp.dot(h, wlv[...], preferred_element_type=jnp.float32) + blv[...]
    mu_o[...] = mu
    lv_o[...] = jnp.clip(lv, -20.0, 20.0)
    tuned_o[...] = ini[...] + jnp.where(mu >= 0, mu, 0.25 * mu)


def _final(parts, w1, b1, wmu, bmu, wlv, blv, ini):
    full = pl.BlockSpec((_D, _D), lambda i: (0, 0))
    vec = pl.BlockSpec((1, _D), lambda i: (0, 0))
    blk = pl.BlockSpec((_BLK, _D), lambda i: (i, 0))
    return pl.pallas_call(
        _final_body,
        grid=(_N // _BLK,),
        in_specs=[
            pl.BlockSpec((1, _BLK, _D), lambda i: (0, i, 0)),
            pl.BlockSpec((1, _BLK, _D), lambda i: (1, i, 0)),
            full, vec, full, vec, full, vec, blk,
        ],
        out_specs=(blk, blk, blk),
        out_shape=(
            jax.ShapeDtypeStruct((_N, _D), jnp.float32),
            jax.ShapeDtypeStruct((_N, _D), jnp.float32),
            jax.ShapeDtypeStruct((_N, _D), jnp.float32),
        ),
    )(parts, parts, w1, b1.reshape(1, _D), wmu, bmu.reshape(1, _D),
      wlv, blv.reshape(1, _D), ini)


@jax.jit
def kernel(edge_index, edge_vals, node_feats, ini_embeds,
           W0, b0, W1, b1, Wmu, bmu, Wlv, blv):
    rows = edge_index[0]
    cols = edge_index[1]
    s1 = _spmm(node_feats, rows, cols, edge_vals)
    h1 = _layer(s1, W0, b0)
    s2 = _spmm(h1, rows, cols, edge_vals)
    return _final(s2, W1, b1, Wmu, bmu, Wlv, blv, ini_embeds)
